# baseline (device time: 37459 ns/iter reference)
import jax
import jax.numpy as jnp
from jax import lax
from jax.experimental import pallas as pl
from jax.experimental.pallas import tpu as pltpu

N_DEV = 8
M = 768
H = 1536
OUT = 768
CHUNK = M // N_DEV


def kernel(x, W1, W2):
    def body(x_ref, w1_ref, w2_ref, out_ref,
             part_f32, part_bf, rs_recv,
             send_sems, recv_sems, ag_send_sems, ag_recv_sems):
        my = lax.axis_index("i")

        barrier_sem = pltpu.get_barrier_semaphore()
        for d in range(1, N_DEV):
            pl.semaphore_signal(
                barrier_sem, inc=1,
                device_id=((my + d) % N_DEV,),
                device_id_type=pl.DeviceIdType.MESH,
            )

        w1b = w1_ref[:, :].astype(jnp.bfloat16)
        w2b = w2_ref[:, :].astype(jnp.bfloat16)

        pl.semaphore_wait(barrier_sem, N_DEV - 1)

        for j in range(N_DEV):
            xj = x_ref[pl.ds(j * CHUNK, CHUNK), :].astype(jnp.bfloat16)
            hj = jnp.dot(xj, w1b, preferred_element_type=jnp.float32)
            hj = jnp.maximum(hj, 0.0).astype(jnp.bfloat16)
            p = jnp.dot(hj, w2b, preferred_element_type=jnp.float32)
            part_f32[pl.ds(j * CHUNK, CHUNK), :] = p
            part_bf[pl.ds(j * CHUNK, CHUNK), :] = p.astype(jnp.bfloat16)
            slot = (j - my + N_DEV) % N_DEV - 1

            @pl.when(j != my)
            def _():
                rdma = pltpu.make_async_remote_copy(
                    src_ref=part_bf.at[pl.ds(j * CHUNK, CHUNK), :],
                    dst_ref=rs_recv.at[slot],
                    send_sem=send_sems.at[slot],
                    recv_sem=recv_sems.at[slot],
                    device_id=(j,),
                    device_id_type=pl.DeviceIdType.MESH,
                )
                rdma.start()

        def _rs_descriptor(k):
            return pltpu.make_async_remote_copy(
                src_ref=part_bf.at[pl.ds(my * CHUNK, CHUNK), :],
                dst_ref=rs_recv.at[k],
                send_sem=send_sems.at[k],
                recv_sem=recv_sems.at[k],
                device_id=(my,),
                device_id_type=pl.DeviceIdType.MESH,
            )

        red = part_f32[pl.ds(my * CHUNK, CHUNK), :]
        for k in range(N_DEV - 1):
            _rs_descriptor(k).wait_recv()
            red = red + rs_recv[k, :, :].astype(jnp.float32)
        out_ref[pl.ds(my * CHUNK, CHUNK), :] = red.astype(jnp.bfloat16)

        ag_sends = []
        for d in range(1, N_DEV):
            t = (my + d) % N_DEV
            rdma = pltpu.make_async_remote_copy(
                src_ref=out_ref.at[pl.ds(my * CHUNK, CHUNK), :],
                dst_ref=out_ref.at[pl.ds(my * CHUNK, CHUNK), :],
                send_sem=ag_send_sems.at[d - 1],
                recv_sem=ag_recv_sems.at[d - 1],
                device_id=(t,),
                device_id_type=pl.DeviceIdType.MESH,
            )
            rdma.start()
            ag_sends.append(rdma)

        for k in range(N_DEV - 1):
            _rs_descriptor(k).wait_send()

        for d in range(1, N_DEV):
            ag_sends[d - 1].wait_recv()
        for d in range(1, N_DEV):
            ag_sends[d - 1].wait_send()

    out_shape = jax.ShapeDtypeStruct((M, OUT), jnp.bfloat16)
    return pl.pallas_call(
        body,
        out_shape=out_shape,
        in_specs=[
            pl.BlockSpec(memory_space=pltpu.VMEM),
            pl.BlockSpec(memory_space=pltpu.VMEM),
            pl.BlockSpec(memory_space=pltpu.VMEM),
        ],
        out_specs=pl.BlockSpec(memory_space=pltpu.VMEM),
        scratch_shapes=[
            pltpu.VMEM((M, OUT), jnp.float32),
            pltpu.VMEM((M, OUT), jnp.bfloat16),
            pltpu.VMEM((N_DEV - 1, CHUNK, OUT), jnp.bfloat16),
            pltpu.SemaphoreType.DMA((N_DEV - 1,)),
            pltpu.SemaphoreType.DMA((N_DEV - 1,)),
            pltpu.SemaphoreType.DMA((N_DEV - 1,)),
            pltpu.SemaphoreType.DMA((N_DEV - 1,)),
        ],
        compiler_params=pltpu.CompilerParams(collective_id=0),
    )(x, W1, W2)


# device time: 32552 ns/iter; 1.1507x vs baseline; 1.1507x over previous
import jax
import jax.numpy as jnp
from jax import lax
from jax.experimental import pallas as pl
from jax.experimental.pallas import tpu as pltpu

N_DEV = 8
M = 768
H = 1536
OUT = 768
CHUNK = M // N_DEV


def kernel(x, W1, W2):
    def body(x_ref, w1_ref, w2_ref, out_ref,
             h_bf, part_f32, part_bf, rs_recv,
             send_sems, recv_sems, ag_send_sems, ag_recv_sems):
        my = lax.axis_index("i")

        barrier_sem = pltpu.get_barrier_semaphore()
        for d in range(1, N_DEV):
            pl.semaphore_signal(
                barrier_sem, inc=1,
                device_id=((my + d) % N_DEV,),
                device_id_type=pl.DeviceIdType.MESH,
            )

        xb = x_ref[:, :].astype(jnp.bfloat16)
        w1b = w1_ref[:, :].astype(jnp.bfloat16)
        h = jnp.dot(xb, w1b, preferred_element_type=jnp.float32)
        h_bf[:, :] = jnp.maximum(h, 0.0).astype(jnp.bfloat16)
        w2b = w2_ref[:, :].astype(jnp.bfloat16)

        pl.semaphore_wait(barrier_sem, N_DEV - 1)

        for j in range(N_DEV):
            p = jnp.dot(h_bf[pl.ds(j * CHUNK, CHUNK), :], w2b,
                        preferred_element_type=jnp.float32)
            part_f32[pl.ds(j * CHUNK, CHUNK), :] = p
            part_bf[pl.ds(j * CHUNK, CHUNK), :] = p.astype(jnp.bfloat16)
            slot = (j - my + N_DEV) % N_DEV - 1

            @pl.when(j != my)
            def _():
                rdma = pltpu.make_async_remote_copy(
                    src_ref=part_bf.at[pl.ds(j * CHUNK, CHUNK), :],
                    dst_ref=rs_recv.at[slot],
                    send_sem=send_sems.at[slot],
                    recv_sem=recv_sems.at[slot],
                    device_id=(j,),
                    device_id_type=pl.DeviceIdType.MESH,
                )
                rdma.start()

        def _rs_descriptor(k):
            return pltpu.make_async_remote_copy(
                src_ref=part_bf.at[pl.ds(my * CHUNK, CHUNK), :],
                dst_ref=rs_recv.at[k],
                send_sem=send_sems.at[k],
                recv_sem=recv_sems.at[k],
                device_id=(my,),
                device_id_type=pl.DeviceIdType.MESH,
            )

        red = part_f32[pl.ds(my * CHUNK, CHUNK), :]
        for k in range(N_DEV - 1):
            _rs_descriptor(k).wait_recv()
            red = red + rs_recv[k, :, :].astype(jnp.float32)
        out_ref[pl.ds(my * CHUNK, CHUNK), :] = red.astype(jnp.bfloat16)

        ag_sends = []
        for d in range(1, N_DEV):
            t = (my + d) % N_DEV
            rdma = pltpu.make_async_remote_copy(
                src_ref=out_ref.at[pl.ds(my * CHUNK, CHUNK), :],
                dst_ref=out_ref.at[pl.ds(my * CHUNK, CHUNK), :],
                send_sem=ag_send_sems.at[d - 1],
                recv_sem=ag_recv_sems.at[d - 1],
                device_id=(t,),
                device_id_type=pl.DeviceIdType.MESH,
            )
            rdma.start()
            ag_sends.append(rdma)

        for k in range(N_DEV - 1):
            _rs_descriptor(k).wait_send()

        for d in range(1, N_DEV):
            ag_sends[d - 1].wait_recv()
        for d in range(1, N_DEV):
            ag_sends[d - 1].wait_send()

    out_shape = jax.ShapeDtypeStruct((M, OUT), jnp.bfloat16)
    return pl.pallas_call(
        body,
        out_shape=out_shape,
        in_specs=[
            pl.BlockSpec(memory_space=pltpu.VMEM),
            pl.BlockSpec(memory_space=pltpu.VMEM),
            pl.BlockSpec(memory_space=pltpu.VMEM),
        ],
        out_specs=pl.BlockSpec(memory_space=pltpu.VMEM),
        scratch_shapes=[
            pltpu.VMEM((M, H), jnp.bfloat16),
            pltpu.VMEM((M, OUT), jnp.float32),
            pltpu.VMEM((M, OUT), jnp.bfloat16),
            pltpu.VMEM((N_DEV - 1, CHUNK, OUT), jnp.bfloat16),
            pltpu.SemaphoreType.DMA((N_DEV - 1,)),
            pltpu.SemaphoreType.DMA((N_DEV - 1,)),
            pltpu.SemaphoreType.DMA((N_DEV - 1,)),
            pltpu.SemaphoreType.DMA((N_DEV - 1,)),
        ],
        compiler_params=pltpu.CompilerParams(collective_id=0),
    )(x, W1, W2)
